# baseline (speedup 1.0000x reference)
import functools

import jax
import jax.numpy as jnp
from jax import lax
from jax.experimental import pallas as pl
from jax.experimental.pallas import tpu as pltpu
from jax.experimental.pallas import tpu_sc as plsc

VOCAB = 1000000
DIM = 64
WIDE = 2 * DIM
BATCH = 16384
NUM_NEG = 10
NUM_ROWS = NUM_NEG + 2
PAR_ROWS = 16

NC = 2
NS = 16
NW = NC * NS
CHUNK = BATCH // NW


SUB = 256
NCHUNK = NUM_ROWS * CHUNK // SUB
PER_W = NUM_ROWS * CHUNK


def _chunk_dst(wid, c):
    j, h = divmod(c, CHUNK // SUB)
    return j * BATCH + wid * CHUNK + h * SUB


def _sc_gather(in_wide, out_wide, idx_flat):
    mesh = plsc.VectorSubcoreMesh(core_axis_name="c", subcore_axis_name="s")

    @functools.partial(
        pl.kernel,
        mesh=mesh,
        out_type=jax.ShapeDtypeStruct((NUM_ROWS * BATCH, WIDE), jnp.float32),
        scratch_types=[
            pltpu.VMEM((PER_W,), jnp.int32),
            pltpu.VMEM((SUB, WIDE), jnp.float32),
            pltpu.VMEM((SUB, WIDE), jnp.float32),
            pltpu.SemaphoreType.DMA,
            pltpu.SemaphoreType.DMA,
        ],
    )
    def gather_kernel(in_hbm, out_hbm, idx_hbm, g_hbm, idx_v, rows_a, rows_b,
                      sem_a, sem_b):
        wid = lax.axis_index("s") * NC + lax.axis_index("c")
        pltpu.sync_copy(idx_hbm.at[pl.ds(wid * PER_W, PER_W)], idx_v)
        bufs = (rows_a, rows_b)
        sems = (sem_a, sem_b)
        pend = [None, None]
        for c in range(NCHUNK):
            s = c % 2
            table = in_hbm if c < CHUNK // SUB else out_hbm
            pend[s] = pltpu.async_copy(
                table.at[idx_v.at[pl.ds(c * SUB, SUB)]], bufs[s], sems[s]
            )
            if c > 0:
                pend[1 - s].wait()
                pltpu.sync_copy(
                    bufs[1 - s], g_hbm.at[pl.ds(_chunk_dst(wid, c - 1), SUB)]
                )
        s = (NCHUNK - 1) % 2
        pend[s].wait()
        pltpu.sync_copy(
            bufs[s], g_hbm.at[pl.ds(_chunk_dst(wid, NCHUNK - 1), SUB)]
        )

    return gather_kernel(in_wide, out_wide, idx_flat)


BB = 1024


def _loss_kernel(g_ref, p_ref, o_ref):
    i = pl.program_id(0)
    g = g_ref[...]
    par = p_ref[...]
    sel = par[:NUM_ROWS, :, None] > 0.5
    rows = jnp.where(sel, g[:, :, DIM:], g[:, :, :DIM])
    center = rows[0]
    scores = jnp.sum(center[None, :, :] * rows[1:], axis=-1)

    def softplus(x):
        return jnp.maximum(x, 0.0) + jnp.log1p(jnp.exp(-jnp.abs(x)))

    block = jnp.sum(softplus(-scores[0])) + jnp.sum(softplus(scores[1:]))

    @pl.when(i == 0)
    def _():
        o_ref[...] = jnp.zeros_like(o_ref)

    o_ref[...] += block


def _tc_loss(gathered, parity):
    g3 = gathered.reshape(NUM_ROWS, BATCH, WIDE)
    nb = BATCH // BB
    out = pl.pallas_call(
        _loss_kernel,
        grid=(nb,),
        in_specs=[
            pl.BlockSpec((NUM_ROWS, BB, WIDE), lambda i: (0, i, 0)),
            pl.BlockSpec((PAR_ROWS, BB), lambda i: (0, i)),
        ],
        out_specs=pl.BlockSpec((1, 1), lambda i: (0, 0)),
        out_shape=jax.ShapeDtypeStruct((1, 1), jnp.float32),
    )(g3, parity)
    return out[0, 0] / BATCH


def kernel(center_nodes, context_nodes, negative_nodes, input_emb, output_emb):
    idx = jnp.concatenate(
        [
            center_nodes.astype(jnp.int32)[None, :],
            context_nodes.astype(jnp.int32)[None, :],
            negative_nodes.astype(jnp.int32).T,
        ],
        axis=0,
    )
    parity = jnp.zeros((PAR_ROWS, BATCH), jnp.float32)
    parity = parity.at[:NUM_ROWS].set((idx & 1).astype(jnp.float32))
    idx_half = (
        (idx >> 1).reshape(NUM_ROWS, NW, CHUNK).transpose(1, 0, 2).reshape(-1)
    )
    in_wide = input_emb.reshape(VOCAB // 2, WIDE)
    out_wide = output_emb.reshape(VOCAB // 2, WIDE)
    gathered = _sc_gather(in_wide, out_wide, idx_half)
    return _tc_loss(gathered, parity)


# speedup vs baseline: 1.2028x; 1.2028x over previous
import functools

import jax
import jax.numpy as jnp
from jax import lax
from jax.experimental import pallas as pl
from jax.experimental.pallas import tpu as pltpu
from jax.experimental.pallas import tpu_sc as plsc

VOCAB = 1000000
DIM = 64
WIDE = 128
BATCH = 16384
NUM_NEG = 10
NUM_ROWS = NUM_NEG + 2
NUM_CN = NUM_NEG + 1
PAR_ROWS = 16

NC = 2
NS = 16
NW = NC * NS
SUB = 256

TL = 2048
NWB = (VOCAB + TL - 1) // TL
WROWS = NWB * (TL // 2)


def _relayout_kernel(t_ref, o_ref):
    blk = t_ref[...]
    o_ref[:, :DIM] = blk[:, : TL // 2].T
    o_ref[:, DIM:] = blk[:, TL // 2:].T


def _relayout(table_t):
    return pl.pallas_call(
        _relayout_kernel,
        grid=(NWB,),
        in_specs=[pl.BlockSpec((DIM, TL), lambda i: (0, i))],
        out_specs=pl.BlockSpec((TL // 2, WIDE), lambda i: (i, 0)),
        out_shape=jax.ShapeDtypeStruct((WROWS, WIDE), jnp.float32),
    )(table_t)


def _sc_gather(table_wide, idx_flat, n_rows):
    per_w = n_rows * BATCH // NW
    nchunk = per_w // SUB
    chunk_b = BATCH // NW
    mesh = plsc.VectorSubcoreMesh(core_axis_name="c", subcore_axis_name="s")

    @functools.partial(
        pl.kernel,
        mesh=mesh,
        out_type=jax.ShapeDtypeStruct((n_rows * BATCH, WIDE), jnp.float32),
        scratch_types=[
            pltpu.VMEM((per_w,), jnp.int32),
            pltpu.VMEM((SUB, WIDE), jnp.float32),
            pltpu.VMEM((SUB, WIDE), jnp.float32),
            pltpu.SemaphoreType.DMA,
            pltpu.SemaphoreType.DMA,
        ],
    )
    def gather_kernel(t_hbm, idx_hbm, g_hbm, idx_v, rows_a, rows_b,
                      sem_a, sem_b):
        wid = lax.axis_index("s") * NC + lax.axis_index("c")
        pltpu.sync_copy(idx_hbm.at[pl.ds(wid * per_w, per_w)], idx_v)
        bufs = (rows_a, rows_b)
        sems = (sem_a, sem_b)
        pend = [None, None]

        def dst(c):
            j, h = divmod(c, chunk_b // SUB)
            return j * BATCH + wid * chunk_b + h * SUB

        for c in range(nchunk):
            s = c % 2
            pend[s] = pltpu.async_copy(
                t_hbm.at[idx_v.at[pl.ds(c * SUB, SUB)]], bufs[s], sems[s]
            )
            if c > 0:
                pend[1 - s].wait()
                pltpu.sync_copy(
                    bufs[1 - s], g_hbm.at[pl.ds(dst(c - 1), SUB)]
                )
        s = (nchunk - 1) % 2
        pend[s].wait()
        pltpu.sync_copy(bufs[s], g_hbm.at[pl.ds(dst(nchunk - 1), SUB)])

    return gather_kernel(table_wide, idx_flat)


BB = 1024


def _loss_kernel(c_ref, n_ref, p_ref, o_ref):
    i = pl.program_id(0)
    c = c_ref[...]
    n = n_ref[...]
    par = p_ref[...]
    csel = jnp.where(par[0][:, None] > 0.5, c[:, DIM:], c[:, :DIM])
    nsel = jnp.where(par[1:NUM_ROWS][:, :, None] > 0.5,
                     n[:, :, DIM:], n[:, :, :DIM])
    scores = jnp.sum(csel[None, :, :] * nsel, axis=-1)

    def softplus(x):
        return jnp.maximum(x, 0.0) + jnp.log1p(jnp.exp(-jnp.abs(x)))

    block = jnp.sum(softplus(-scores[0])) + jnp.sum(softplus(scores[1:]))

    @pl.when(i == 0)
    def _():
        o_ref[...] = jnp.zeros_like(o_ref)

    o_ref[...] += block


def _tc_loss(g_center, g_cn, parity):
    n3 = g_cn.reshape(NUM_CN, BATCH, WIDE)
    nb = BATCH // BB
    out = pl.pallas_call(
        _loss_kernel,
        grid=(nb,),
        in_specs=[
            pl.BlockSpec((BB, WIDE), lambda i: (i, 0)),
            pl.BlockSpec((NUM_CN, BB, WIDE), lambda i: (0, i, 0)),
            pl.BlockSpec((PAR_ROWS, BB), lambda i: (0, i)),
        ],
        out_specs=pl.BlockSpec((1, 1), lambda i: (0, 0)),
        out_shape=jax.ShapeDtypeStruct((1, 1), jnp.float32),
    )(g_center, n3, parity)
    return out[0, 0] / BATCH


def kernel(center_nodes, context_nodes, negative_nodes, input_emb, output_emb):
    idx = jnp.concatenate(
        [
            center_nodes.astype(jnp.int32)[None, :],
            context_nodes.astype(jnp.int32)[None, :],
            negative_nodes.astype(jnp.int32).T,
        ],
        axis=0,
    )
    r = idx % TL
    sel = (r >= TL // 2).astype(jnp.float32)
    parity = jnp.zeros((PAR_ROWS, BATCH), jnp.float32)
    parity = parity.at[:NUM_ROWS].set(sel)
    half = (idx // TL) * (TL // 2) + (r % (TL // 2))
    idx_cn = (
        half[1:].reshape(NUM_CN, NW, BATCH // NW)
        .transpose(1, 0, 2).reshape(-1)
    )
    idx_c = half[0].reshape(NW, BATCH // NW).reshape(-1)
    out_wide = _relayout(output_emb.T)
    g_cn = _sc_gather(out_wide, idx_cn, NUM_CN)
    in_wide = _relayout(input_emb.T)
    g_center = _sc_gather(in_wide, idx_c, 1)
    return _tc_loss(g_center, g_cn, parity)
